# baseline (speedup 1.0000x reference)
import functools

import jax
import jax.numpy as jnp
from jax import lax
from jax.experimental import pallas as pl
from jax.experimental.pallas import tpu as pltpu

_LOG2E = 1.4426950408889634


def _fused_mha_kernel(x_ref, wqkv_ref, bqkv_ref, wproj_ref, bproj_ref,
                      o_ref, *, num_heads, head_dim, scale, bb, n):
    D = num_heads * head_dim
    xb = x_ref[...].reshape(bb * n, D).astype(jnp.bfloat16)

    qkv = jnp.dot(xb, wqkv_ref[...],
                  preferred_element_type=jnp.float32)
    qkv = qkv + bqkv_ref[...]

    qb = (qkv[:, 0:D] * (scale * _LOG2E)).astype(jnp.bfloat16)
    kb = qkv[:, D:2 * D].astype(jnp.bfloat16)
    vb = qkv[:, 2 * D:3 * D]

    RT = min(128, n)
    row_tiles = []
    for b in range(bb):
        r0 = b * n
        for rt in range(n // RT):
            q0 = r0 + rt * RT
            tile_outs = []
            for h in range(num_heads):
                lo = h * head_dim
                hi = lo + head_dim
                qh = qb[q0:q0 + RT, lo:hi]
                kh = kb[r0:r0 + n, lo:hi]
                vh = vb[r0:r0 + n, lo:hi]
                s = lax.dot_general(qh, kh, (((1,), (1,)), ((), ())),
                                    preferred_element_type=jnp.float32)
                s = s - jnp.max(s, axis=-1, keepdims=True)
                p = jnp.exp2(s)
                l = jnp.sum(p, axis=-1, keepdims=True)
                o = jnp.dot(p, vh,
                            preferred_element_type=jnp.float32)
                tile_outs.append(o * (1.0 / l))
            row_tiles.append(jnp.concatenate(tile_outs, axis=-1))

    y = jnp.concatenate(row_tiles, axis=0).astype(jnp.bfloat16)
    out = jnp.dot(y, wproj_ref[...],
                  preferred_element_type=jnp.float32) + bproj_ref[...]
    o_ref[...] = out.reshape(bb, n, D).astype(o_ref.dtype)


def kernel(x, w_qkv_t, w_proj_t, b_qkv, b_proj):
    B, N, D = x.shape
    num_heads = 12
    head_dim = D // num_heads
    scale = head_dim ** (-0.5)
    BB = 8

    wq = w_qkv_t.astype(jnp.bfloat16)
    wp = w_proj_t.astype(jnp.bfloat16)
    bq = b_qkv.reshape(1, 3 * D)
    bp = b_proj.reshape(1, D)

    kern = functools.partial(_fused_mha_kernel, num_heads=num_heads,
                             head_dim=head_dim, scale=scale, bb=BB, n=N)
    return pl.pallas_call(
        kern,
        out_shape=jax.ShapeDtypeStruct((B, N, D), x.dtype),
        grid=(B // BB,),
        in_specs=[
            pl.BlockSpec((BB, N, D), lambda b: (b, 0, 0)),
            pl.BlockSpec((D, 3 * D), lambda b: (0, 0)),
            pl.BlockSpec((1, 3 * D), lambda b: (0, 0)),
            pl.BlockSpec((D, D), lambda b: (0, 0)),
            pl.BlockSpec((1, D), lambda b: (0, 0)),
        ],
        out_specs=pl.BlockSpec((BB, N, D), lambda b: (b, 0, 0)),
        compiler_params=pltpu.CompilerParams(
            dimension_semantics=("parallel",),
            allow_input_fusion=[False, True, False, True, False],
            vmem_limit_bytes=100 * 1024 * 1024),
    )(x, wq, bq, wp, bp)


# speedup vs baseline: 1.1221x; 1.1221x over previous
import functools

import jax
import jax.numpy as jnp
from jax import lax
from jax.experimental import pallas as pl
from jax.experimental.pallas import tpu as pltpu

_LOG2E = 1.4426950408889634


def _fused_mha_kernel(x_ref, wqkv_ref, bqkv_ref, wproj_ref, bproj_ref,
                      o_ref, *, num_heads, head_dim, scale, bb, n):
    D = num_heads * head_dim
    xb = x_ref[...].reshape(bb * n, D).astype(jnp.bfloat16)

    qkv = jnp.dot(xb, wqkv_ref[...],
                  preferred_element_type=jnp.float32)
    qkv = qkv + bqkv_ref[...]

    qb = (qkv[:, 0:D] * (scale * _LOG2E)).astype(jnp.bfloat16)
    kb = qkv[:, D:2 * D].astype(jnp.bfloat16)
    vb = qkv[:, 2 * D:3 * D]

    RT = min(128, n)
    row_tiles = []
    for b in range(bb):
        r0 = b * n
        for rt in range(n // RT):
            q0 = r0 + rt * RT
            tile_outs = []
            for h in range(num_heads):
                lo = h * head_dim
                hi = lo + head_dim
                qh = qb[q0:q0 + RT, lo:hi]
                kh = kb[r0:r0 + n, lo:hi]
                vh = vb[r0:r0 + n, lo:hi]
                s = lax.dot_general(qh, kh, (((1,), (1,)), ((), ())),
                                    preferred_element_type=jnp.float32)
                s = s - jnp.max(s, axis=-1, keepdims=True)
                p = jnp.exp2(s)
                l = jnp.sum(p, axis=-1, keepdims=True)
                o = jnp.dot(p, vh,
                            preferred_element_type=jnp.float32)
                tile_outs.append(o * (1.0 / l))
            row_tiles.append(jnp.concatenate(tile_outs, axis=-1))

    y = jnp.concatenate(row_tiles, axis=0).astype(jnp.bfloat16)
    out = jnp.dot(y, wproj_ref[...],
                  preferred_element_type=jnp.float32) + bproj_ref[...]
    o_ref[...] = out.reshape(bb, n, D).astype(o_ref.dtype)


def kernel(x, w_qkv_t, w_proj_t, b_qkv, b_proj):
    B, N, D = x.shape
    num_heads = 12
    head_dim = D // num_heads
    scale = head_dim ** (-0.5)
    BB = 4

    wq = w_qkv_t.astype(jnp.bfloat16)
    wp = w_proj_t.astype(jnp.bfloat16)
    bq = b_qkv.reshape(1, 3 * D)
    bp = b_proj.reshape(1, D)

    kern = functools.partial(_fused_mha_kernel, num_heads=num_heads,
                             head_dim=head_dim, scale=scale, bb=BB, n=N)
    return pl.pallas_call(
        kern,
        out_shape=jax.ShapeDtypeStruct((B, N, D), x.dtype),
        grid=(B // BB,),
        in_specs=[
            pl.BlockSpec((BB, N, D), lambda b: (b, 0, 0)),
            pl.BlockSpec((D, 3 * D), lambda b: (0, 0)),
            pl.BlockSpec((1, 3 * D), lambda b: (0, 0)),
            pl.BlockSpec((D, D), lambda b: (0, 0)),
            pl.BlockSpec((1, D), lambda b: (0, 0)),
        ],
        out_specs=pl.BlockSpec((BB, N, D), lambda b: (b, 0, 0)),
        compiler_params=pltpu.CompilerParams(
            dimension_semantics=("parallel",),
            allow_input_fusion=[False, True, False, True, False],
            vmem_limit_bytes=100 * 1024 * 1024),
    )(x, wq, bq, wp, bp)
